# baseline (speedup 1.0000x reference)
import jax
import jax.numpy as jnp
from jax.experimental import pallas as pl
from jax.experimental.pallas import tpu as pltpu

B, S, D, A, E, K = 8, 128, 128, 128, 8, 2
N = B * S
NS = 2
CHS = 8192
CH = NS * CHS
NC = (D * D) // CH


def _gating_body(ad_ref, gw_ref, gb_ref, w_ref, awb_ref, abw_ref, abb_ref,
                 mask_be_ref, ord_ref, bal_ref, wsum_ref, bias_ref):
    ad = ad_ref[...]
    gw = gw_ref[...]
    st = jax.lax.dot_general(ad, gw, (((1,), (1,)), ((), ())),
                             preferred_element_type=jnp.float32)
    st = st + gb_ref[...]
    ioe = jax.lax.broadcasted_iota(jnp.int32, (B, E), 1)
    neg = jnp.float32(-1e30)
    m1 = jnp.max(st, axis=1, keepdims=True)
    idx1 = jnp.min(jnp.where(st >= m1, ioe, jnp.int32(E)), axis=1,
                   keepdims=True)
    oh1 = ioe == idx1
    st2 = jnp.where(oh1, neg, st)
    m2 = jnp.max(st2, axis=1, keepdims=True)
    idx2 = jnp.min(jnp.where(st2 >= m2, ioe, jnp.int32(E)), axis=1,
                   keepdims=True)
    mask_be = (oh1 | (ioe == idx2)).astype(jnp.float32)
    mask_be_ref[...] = mask_be
    eye = (jax.lax.broadcasted_iota(jnp.int32, (E, E), 0) ==
           jax.lax.broadcasted_iota(jnp.int32, (E, E), 1)).astype(jnp.float32)
    mask_eb = jax.lax.dot_general(eye, mask_be, (((1,), (1,)), ((), ())),
                                  preferred_element_type=jnp.float32)
    cnt = jnp.sum(mask_eb, axis=1, keepdims=True)
    act_col = (cnt > 0).astype(jnp.float32)
    usage = cnt * jnp.float32(S)
    bal = jnp.sum((usage - jnp.float32(N * K / E)) ** 2)
    bal_ref[...] = jnp.reshape(bal, (1, 1))

    wsum_ref[...] = jax.lax.dot_general(
        mask_be, w_ref[...] + awb_ref[...], (((1,), (0,)), ((), ())),
        preferred_element_type=jnp.float32)

    abw_eff = jnp.sum(abw_ref[...] * act_col[:, :, None], axis=0)
    abb_eff = jnp.sum(abb_ref[...] * act_col, axis=0,
                      keepdims=True)
    bias = jax.lax.dot_general(ad, abw_eff, (((1,), (1,)), ((), ())),
                               preferred_element_type=jnp.float32)
    bias_ref[...] = bias + abb_eff

    ioe_sq = jax.lax.broadcasted_iota(jnp.int32, (E, E), 1)
    iob_sq = jax.lax.broadcasted_iota(jnp.int32, (E, E), 0)
    lt = (ioe_sq < iob_sq).astype(jnp.float32)
    rank = jax.lax.dot_general(lt, act_col, (((1,), (0,)), ((), ())),
                               preferred_element_type=jnp.float32)
    iok = jax.lax.broadcasted_iota(jnp.int32, (E, E + 1), 1).astype(
        jnp.float32)
    ioe_col = jax.lax.broadcasted_iota(jnp.int32, (E, E + 1), 0).astype(
        jnp.float32)
    ind = ((act_col > 0) & (rank == iok)).astype(jnp.float32)
    base = jnp.max(jnp.where(ind > 0, ioe_col, -1.0), axis=0,
                   keepdims=True)
    n_active = jnp.sum(act_col)
    last_active = jnp.max(
        jnp.where(act_col > 0,
                  jax.lax.broadcasted_iota(jnp.int32, (E, 1), 0).astype(
                      jnp.float32), -1.0))
    iok_row = jax.lax.broadcasted_iota(jnp.int32, (1, E + 1), 1).astype(
        jnp.float32)
    order = jnp.where(iok_row >= n_active, last_active, base)
    order = jnp.where(iok_row == jnp.float32(E), n_active, order)
    ord_ref[...] = order.astype(jnp.int32)


def _madapt_body(ord_ref, *refs):
    aw_refs, (ad_ref, mask_be_ref, out_ref) = refs[:NS], refs[NS:]
    i = pl.program_id(1)
    expert = ord_ref[i]
    valid = (i < ord_ref[E]).astype(jnp.float32)
    ioe = jax.lax.broadcasted_iota(jnp.int32, (B, E), 1)
    mcol = jnp.sum(jnp.where(ioe == expert, mask_be_ref[...], 0.0), axis=1,
                   keepdims=True) * valid
    adm = ad_ref[...] * mcol
    contribs = [
        jax.lax.dot_general(adm, r[0, 0, 0], (((1,), (1,)), ((), ())),
                            preferred_element_type=jnp.float32)
        for r in aw_refs
    ]
    contrib = jnp.concatenate(contribs, axis=1)

    @pl.when(i == 0)
    def _():
        out_ref[...] = contrib

    @pl.when(i > 0)
    def _():
        out_ref[...] += contrib


def _combine_body(x_ref, mad_ref, wsum_ref, bias_ref, out_ref):
    m_b = mad_ref[0] + wsum_ref[0]
    out = jax.lax.dot_general(x_ref[0], m_b, (((1,), (1,)), ((), ())),
                              preferred_element_type=jnp.float32)
    out_ref[0] = out + bias_ref[0]


def kernel(x, adapt_input, W, b, AW_w, AW_b, AB_w, AB_b, G_w, G_b):
    f32 = jnp.float32
    mask_be, ordv, bal, wsum, bias = pl.pallas_call(
        _gating_body,
        out_shape=[
            jax.ShapeDtypeStruct((B, E), f32),
            jax.ShapeDtypeStruct((1, E + 1), jnp.int32),
            jax.ShapeDtypeStruct((1, 1), f32),
            jax.ShapeDtypeStruct((B, D * D), f32),
            jax.ShapeDtypeStruct((B, D), f32),
        ],
    )(adapt_input, G_w, G_b.reshape(1, E), W.reshape(E, D * D),
      AW_b, AB_w, AB_b)

    awr = AW_w.reshape(E, NC, NS, CHS, A)

    def _aw_spec(s):
        return pl.BlockSpec((1, 1, 1, CHS, A),
                            lambda c, i, ordr, s=s: (ordr[i], c, s, 0, 0))

    mad = pl.pallas_call(
        _madapt_body,
        grid_spec=pltpu.PrefetchScalarGridSpec(
            num_scalar_prefetch=1,
            grid=(NC, E),
            in_specs=[_aw_spec(s) for s in range(NS)] + [
                pl.BlockSpec((B, A), lambda c, i, ordr: (0, 0)),
                pl.BlockSpec((B, E), lambda c, i, ordr: (0, 0)),
            ],
            out_specs=pl.BlockSpec((B, CH), lambda c, i, ordr: (0, c)),
        ),
        out_shape=jax.ShapeDtypeStruct((B, D * D), f32),
        compiler_params=pltpu.CompilerParams(
            dimension_semantics=("arbitrary", "arbitrary")),
    )(ordv.reshape(E + 1), *([awr] * NS), adapt_input, mask_be)

    out = pl.pallas_call(
        _combine_body,
        grid=(B,),
        in_specs=[
            pl.BlockSpec((1, S, D), lambda bb: (bb, 0, 0)),
            pl.BlockSpec((1, D, D), lambda bb: (bb, 0, 0)),
            pl.BlockSpec((1, D, D), lambda bb: (bb, 0, 0)),
            pl.BlockSpec((1, 1, D), lambda bb: (bb, 0, 0)),
        ],
        out_specs=pl.BlockSpec((1, S, D), lambda bb: (bb, 0, 0)),
        out_shape=jax.ShapeDtypeStruct((B, S, D), f32),
        compiler_params=pltpu.CompilerParams(
            dimension_semantics=("parallel",)),
    )(x, mad.reshape(B, D, D), wsum.reshape(B, D, D), bias.reshape(B, 1, D))

    return out, bal.reshape(())


# speedup vs baseline: 1.0976x; 1.0976x over previous
import jax
import jax.numpy as jnp
from jax.experimental import pallas as pl
from jax.experimental.pallas import tpu as pltpu

B, S, D, A, E, K = 8, 128, 128, 128, 8, 2
N = B * S
NS = 2
CHS = 8192
CH = NS * CHS
NC = (D * D) // CH


def _gating_body(ad_ref, gw_ref, gb_ref, w_ref, awb_ref, abw_ref, abb_ref,
                 mask_be_ref, ord_ref, bal_ref, wsum_ref, bias_ref):
    ad = ad_ref[...]
    gw = gw_ref[...]
    st = jax.lax.dot_general(ad, gw, (((1,), (1,)), ((), ())),
                             preferred_element_type=jnp.float32)
    st = st + gb_ref[...]
    ioe = jax.lax.broadcasted_iota(jnp.int32, (B, E), 1)
    neg = jnp.float32(-1e30)
    m1 = jnp.max(st, axis=1, keepdims=True)
    idx1 = jnp.min(jnp.where(st >= m1, ioe, jnp.int32(E)), axis=1,
                   keepdims=True)
    oh1 = ioe == idx1
    st2 = jnp.where(oh1, neg, st)
    m2 = jnp.max(st2, axis=1, keepdims=True)
    idx2 = jnp.min(jnp.where(st2 >= m2, ioe, jnp.int32(E)), axis=1,
                   keepdims=True)
    mask_be = (oh1 | (ioe == idx2)).astype(jnp.float32)
    mask_be_ref[...] = mask_be
    eye = (jax.lax.broadcasted_iota(jnp.int32, (E, E), 0) ==
           jax.lax.broadcasted_iota(jnp.int32, (E, E), 1)).astype(jnp.float32)
    mask_eb = jax.lax.dot_general(eye, mask_be, (((1,), (1,)), ((), ())),
                                  preferred_element_type=jnp.float32)
    cnt = jnp.sum(mask_eb, axis=1, keepdims=True)
    act_col = (cnt > 0).astype(jnp.float32)
    usage = cnt * jnp.float32(S)
    bal = jnp.sum((usage - jnp.float32(N * K / E)) ** 2)
    bal_ref[...] = jnp.reshape(bal, (1, 1))

    wsum_ref[...] = jax.lax.dot_general(
        mask_be, w_ref[...] + awb_ref[...], (((1,), (0,)), ((), ())),
        preferred_element_type=jnp.float32)

    abw_eff = jnp.sum(abw_ref[...] * act_col[:, :, None], axis=0)
    abb_eff = jnp.sum(abb_ref[...] * act_col, axis=0,
                      keepdims=True)
    bias = jax.lax.dot_general(ad, abw_eff, (((1,), (1,)), ((), ())),
                               preferred_element_type=jnp.float32)
    bias_ref[...] = bias + abb_eff

    ioe_sq = jax.lax.broadcasted_iota(jnp.int32, (E, E), 1)
    iob_sq = jax.lax.broadcasted_iota(jnp.int32, (E, E), 0)
    lt = (ioe_sq < iob_sq).astype(jnp.float32)
    rank = jax.lax.dot_general(lt, act_col, (((1,), (0,)), ((), ())),
                               preferred_element_type=jnp.float32)
    iok = jax.lax.broadcasted_iota(jnp.int32, (E, E + 1), 1).astype(
        jnp.float32)
    ioe_col = jax.lax.broadcasted_iota(jnp.int32, (E, E + 1), 0).astype(
        jnp.float32)
    ind = ((act_col > 0) & (rank == iok)).astype(jnp.float32)
    base = jnp.max(jnp.where(ind > 0, ioe_col, -1.0), axis=0,
                   keepdims=True)
    n_active = jnp.sum(act_col)
    last_active = jnp.max(
        jnp.where(act_col > 0,
                  jax.lax.broadcasted_iota(jnp.int32, (E, 1), 0).astype(
                      jnp.float32), -1.0))
    iok_row = jax.lax.broadcasted_iota(jnp.int32, (1, E + 1), 1).astype(
        jnp.float32)
    order = jnp.where(iok_row >= n_active, last_active, base)
    order = jnp.where(iok_row == jnp.float32(E), n_active, order)
    ord_ref[...] = order.astype(jnp.int32)


def _madapt_body(ord_ref, *refs):
    aw_refs, (ad_ref, mask_be_ref, out_ref) = refs[:NS], refs[NS:]
    i = pl.program_id(1)
    expert = ord_ref[i]
    valid = (i < ord_ref[E]).astype(jnp.float32)
    ioe = jax.lax.broadcasted_iota(jnp.int32, (B, E), 1)
    mcol = jnp.sum(jnp.where(ioe == expert, mask_be_ref[...], 0.0), axis=1,
                   keepdims=True) * valid
    adm = ad_ref[...] * mcol
    contribs = [
        jax.lax.dot_general(adm, r[0, 0, 0], (((1,), (1,)), ((), ())),
                            preferred_element_type=jnp.float32)
        for r in aw_refs
    ]
    contrib = jnp.concatenate(contribs, axis=1)

    @pl.when(i == 0)
    def _():
        out_ref[...] = contrib

    @pl.when(i > 0)
    def _():
        out_ref[...] += contrib


def _combine_body(x_ref, mad_ref, wsum_ref, bias_ref, out_ref):
    for bb in range(B):
        m_b = mad_ref[bb] + wsum_ref[bb]
        out = jax.lax.dot_general(x_ref[bb], m_b, (((1,), (1,)), ((), ())),
                                  preferred_element_type=jnp.float32)
        out_ref[bb] = out + bias_ref[bb]


def kernel(x, adapt_input, W, b, AW_w, AW_b, AB_w, AB_b, G_w, G_b):
    f32 = jnp.float32
    mask_be, ordv, bal, wsum, bias = pl.pallas_call(
        _gating_body,
        out_shape=[
            jax.ShapeDtypeStruct((B, E), f32),
            jax.ShapeDtypeStruct((1, E + 1), jnp.int32),
            jax.ShapeDtypeStruct((1, 1), f32),
            jax.ShapeDtypeStruct((B, D * D), f32),
            jax.ShapeDtypeStruct((B, D), f32),
        ],
    )(adapt_input, G_w, G_b.reshape(1, E), W.reshape(E, D * D),
      AW_b, AB_w, AB_b)

    awr = AW_w.reshape(E, NC, NS, CHS, A)

    def _aw_spec(s):
        return pl.BlockSpec((1, 1, 1, CHS, A),
                            lambda c, i, ordr, s=s: (ordr[i], c, s, 0, 0))

    mad = pl.pallas_call(
        _madapt_body,
        grid_spec=pltpu.PrefetchScalarGridSpec(
            num_scalar_prefetch=1,
            grid=(NC, E),
            in_specs=[_aw_spec(s) for s in range(NS)] + [
                pl.BlockSpec((B, A), lambda c, i, ordr: (0, 0)),
                pl.BlockSpec((B, E), lambda c, i, ordr: (0, 0)),
            ],
            out_specs=pl.BlockSpec((B, CH), lambda c, i, ordr: (0, c)),
        ),
        out_shape=jax.ShapeDtypeStruct((B, D * D), f32),
        compiler_params=pltpu.CompilerParams(
            dimension_semantics=("arbitrary", "arbitrary")),
    )(ordv.reshape(E + 1), *([awr] * NS), adapt_input, mask_be)

    out = pl.pallas_call(
        _combine_body,
        out_shape=jax.ShapeDtypeStruct((B, S, D), f32),
    )(x, mad.reshape(B, D, D), wsum.reshape(B, D, D), bias.reshape(B, 1, D))

    return out, bal.reshape(())
